# Initial kernel scaffold; baseline (speedup 1.0000x reference)
import jax
import jax.numpy as jnp
from jax.experimental import pallas as pl


def kernel(x, W_in, b_in, W_e0, b_e0, W_s0, b_s0, W_e1, b_e1, W_s1, b_s1, W_out, b_out):
    raise NotImplementedError("write your pallas kernel here")


# speedup vs baseline: 6.9441x; 6.9441x over previous
import functools

import jax
import jax.numpy as jnp
import numpy as np
from jax.experimental import pallas as pl
from jax.experimental.pallas import tpu as pltpu

B = 32
C = 32
F_DIM = 128
H = 32
HH = H * H
E_CNT = C * (C - 1)


def _body(x_ref, W_in_ref, b_in_ref, We0_ref, be0_ref, Ws0_ref, bs0_ref,
          We1_ref, be1_ref, Ws1_ref, bs1_ref, W_out_ref, b_out_ref,
          eye_ref, Emat_ref, Rmat_ref, out_ref, s1_ref):
    f32 = jnp.float32
    xb = x_ref[0]
    h = jnp.maximum(
        jax.lax.dot(xb, W_in_ref[...], preferred_element_type=f32)
        + b_in_ref[...], 0.0)

    G = jax.lax.dot_general(h, h, (((1,), (1,)), ((), ())),
                            preferred_element_type=f32)
    eye = eye_ref[...]
    n2_col = jnp.sum(G * eye, axis=1, keepdims=True)
    n2_row = jnp.sum(G * eye, axis=0, keepdims=True)
    denom = jnp.maximum(jnp.sqrt(n2_col) * jnp.sqrt(n2_row), 1e-8)
    cos = G / denom
    d2 = jnp.maximum(n2_col + n2_row - 2.0 * G, 0.0)
    dist = jnp.sqrt(d2)
    mean_dist = jnp.sum(dist) * (1.0 / E_CNT)
    distn = dist / (mean_dist + 1e-6)

    lane_iota = jax.lax.broadcasted_iota(jnp.int32, (1, C), 1)
    sub_iota = jax.lax.broadcasted_iota(jnp.int32, (C, 1), 0)
    Emat = Emat_ref[...]
    Rmat = Rmat_ref[...]

    def conv(h_in, We_ref, be_ref, Ws_ref, bs_ref):
        We0 = We_ref[0:1, :]
        We1 = We_ref[1:2, :]
        be = be_ref[...]
        h_exp = jax.lax.dot(h_in, Emat, preferred_element_type=f32)

        def jbody(j, _):
            onehot = (lane_iota == j).astype(f32)
            cos_col = jnp.sum(cos * onehot, axis=1, keepdims=True)
            dist_col = jnp.sum(distn * onehot, axis=1, keepdims=True)
            maskc = (sub_iota != j).astype(f32)
            t = jnp.tanh(cos_col * We0 + dist_col * We1 + be)
            row = jnp.sum(t * h_exp * maskc, axis=0, keepdims=True)
            s1_ref[pl.ds(j, 1), :] = row
            return 0

        jax.lax.fori_loop(0, C, jbody, 0, unroll=True)
        s1 = s1_ref[...]
        aggr = jax.lax.dot(s1, Rmat, preferred_element_type=f32) * (1.0 / (C - 1))
        upd = aggr + jax.lax.dot(h_in, Ws_ref[...], preferred_element_type=f32) \
            + bs_ref[...]
        return jnp.maximum(upd, 0.0)

    h1 = conv(h, We0_ref, be0_ref, Ws0_ref, bs0_ref)
    h2 = conv(h1, We1_ref, be1_ref, Ws1_ref, bs1_ref)

    out = jax.lax.dot(h2, W_out_ref[...], preferred_element_type=f32) \
        + b_out_ref[...]
    out_ref[0] = xb + out


@jax.jit
def _run(x, W_in, b_in, W_e0, b_e0, W_s0, b_s0, W_e1, b_e1, W_s1, b_s1,
         W_out, b_out, eyeC, Emat, Rmat):
    full = lambda s: pl.BlockSpec(s, lambda b: (0,) * len(s))
    return pl.pallas_call(
        _body,
        grid=(B,),
        in_specs=[
            pl.BlockSpec((1, C, F_DIM), lambda b: (b, 0, 0)),
            full((F_DIM, H)), full((1, H)),
            full((2, HH)), full((1, HH)),
            full((H, H)), full((1, H)),
            full((2, HH)), full((1, HH)),
            full((H, H)), full((1, H)),
            full((H, F_DIM)), full((1, F_DIM)),
            full((C, C)),
            full((H, HH)),
            full((HH, H)),
        ],
        out_specs=pl.BlockSpec((1, C, F_DIM), lambda b: (b, 0, 0)),
        out_shape=jax.ShapeDtypeStruct((B, C, F_DIM), jnp.float32),
        scratch_shapes=[pltpu.VMEM((C, HH), jnp.float32)],
        compiler_params=pltpu.CompilerParams(
            dimension_semantics=("arbitrary",)),
    )(x, W_in, b_in, W_e0, b_e0, W_s0, b_s0, W_e1, b_e1, W_s1, b_s1,
      W_out, b_out, eyeC, Emat, Rmat)


def kernel(x, W_in, b_in, W_e0, b_e0, W_s0, b_s0, W_e1, b_e1, W_s1, b_s1,
           W_out, b_out):
    eyeC = jnp.asarray(np.eye(C, dtype=np.float32))
    Emat = jnp.asarray(np.kron(np.eye(H), np.ones((1, H))).astype(np.float32))
    Rmat = jnp.asarray(np.tile(np.eye(H), (H, 1)).astype(np.float32))
    return _run(x, W_in, b_in.reshape(1, H), W_e0, b_e0.reshape(1, HH),
                W_s0, b_s0.reshape(1, H), W_e1, b_e1.reshape(1, HH),
                W_s1, b_s1.reshape(1, H), W_out, b_out.reshape(1, F_DIM),
                eyeC, Emat, Rmat)
